# baseline (speedup 1.0000x reference)
import jax
import jax.numpy as jnp
from jax import lax
from jax.experimental import pallas as pl
from jax.experimental.pallas import tpu as pltpu
from jax.experimental.pallas import tpu_sc as plsc

VOCAB = 1000000
SEQ_LEN = 200
EMBED_DIM = 64
BATCH = 4096

NUM_CORES = 2
NUM_SUBCORES = 16
NUM_WORKERS = NUM_CORES * NUM_SUBCORES
SEQ_PER_WORKER = BATCH // NUM_WORKERS
LANES = 16
VREGS_PER_ROW = EMBED_DIM // LANES

GATHER_SPLITS = ((0, 128), (128, 72))

EMBED_SCALE = 8.0


def _sc_kernel_body(inputs_hbm, tok_hbm, pos_hbm, out_hbm,
                    idx_v, rows_v, pos_v, sem):
    wid = lax.axis_index("s") * NUM_CORES + lax.axis_index("c")
    base_row = wid * SEQ_PER_WORKER * SEQ_LEN

    pltpu.sync_copy(pos_hbm, pos_v)

    def seq_body(s, carry):
        row0 = base_row + s * SEQ_LEN
        pltpu.sync_copy(inputs_hbm.at[pl.ds(row0, SEQ_LEN)], idx_v)
        for (off, n) in GATHER_SPLITS:
            pltpu.async_copy(
                tok_hbm.at[idx_v.at[pl.ds(off, n)]],
                rows_v.at[pl.ds(off, n)],
                sem,
            ).wait()

        def row_body(t, c2):
            for c in range(VREGS_PER_ROW):
                sl = pl.ds(c * LANES, LANES)
                rows_v[t, sl] = rows_v[t, sl] * EMBED_SCALE + pos_v[t, sl]
            return c2

        lax.fori_loop(0, SEQ_LEN, row_body, 0, unroll=2)

        pltpu.sync_copy(rows_v, out_hbm.at[pl.ds(row0, SEQ_LEN)])
        return carry

    lax.fori_loop(0, SEQ_PER_WORKER, seq_body, 0)


@jax.jit
def _embed(inputs_flat, tok_table, pos_table):
    mesh = plsc.VectorSubcoreMesh(core_axis_name="c", subcore_axis_name="s")
    fn = pl.kernel(
        _sc_kernel_body,
        out_type=jax.ShapeDtypeStruct((BATCH * SEQ_LEN, EMBED_DIM),
                                      jnp.float32),
        mesh=mesh,
        scratch_types=[
            pltpu.VMEM((SEQ_LEN,), jnp.int32),
            pltpu.VMEM((SEQ_LEN, EMBED_DIM), jnp.float32),
            pltpu.VMEM((SEQ_LEN, EMBED_DIM), jnp.float32),
            pltpu.SemaphoreType.DMA,
        ],
        compiler_params=pltpu.CompilerParams(use_tc_tiling_on_sc=False),
    )
    return fn(inputs_flat, tok_table, pos_table)


def kernel(inputs, tok_table, pos_table):
    flat = inputs.reshape(BATCH * SEQ_LEN)
    out = _embed(flat, tok_table, pos_table)
    return out.reshape(BATCH, SEQ_LEN, EMBED_DIM)


# speedup vs baseline: 1.1749x; 1.1749x over previous
import jax
import jax.numpy as jnp
from jax import lax
from jax.experimental import pallas as pl
from jax.experimental.pallas import tpu as pltpu
from jax.experimental.pallas import tpu_sc as plsc

VOCAB = 1000000
SEQ_LEN = 200
EMBED_DIM = 64
BATCH = 4096

NUM_CORES = 2
NUM_SUBCORES = 16
NUM_WORKERS = NUM_CORES * NUM_SUBCORES
SEQ_PER_WORKER = BATCH // NUM_WORKERS
ROWS_PER_WORKER = SEQ_PER_WORKER * SEQ_LEN
LANES = 16
VREGS_PER_ROW = EMBED_DIM // LANES

GATHER_SPLITS = ((0, 128), (128, 72))

EMBED_SCALE = 8.0


def _sc_kernel_body(inputs_hbm, tok_hbm, pos_hbm, out_hbm,
                    idx_v, rows0, rows1, pos_v,
                    sem_g0, sem_g1, sem_o0, sem_o1):
    wid = lax.axis_index("s") * NUM_CORES + lax.axis_index("c")
    base_row = wid * ROWS_PER_WORKER

    rows = (rows0, rows1)
    sem_g = (sem_g0, sem_g1)
    sem_o = (sem_o0, sem_o1)

    pltpu.sync_copy(inputs_hbm.at[pl.ds(base_row, ROWS_PER_WORKER)], idx_v)
    pltpu.sync_copy(pos_hbm, pos_v)

    def gather(s, b, start):
        for (off, n) in GATHER_SPLITS:
            desc = pltpu.make_async_copy(
                tok_hbm.at[idx_v.at[pl.ds(s * SEQ_LEN + off, n)]],
                rows[b].at[pl.ds(off, n)],
                sem_g[b],
            )
            desc.start() if start else desc.wait()

    def out_copy(s, b, start):
        desc = pltpu.make_async_copy(
            rows[b],
            out_hbm.at[pl.ds(base_row + s * SEQ_LEN, SEQ_LEN)],
            sem_o[b],
        )
        desc.start() if start else desc.wait()

    def compute(b):
        def row_body(t, c2):
            for c in range(VREGS_PER_ROW):
                sl = pl.ds(c * LANES, LANES)
                rows[b][t, sl] = rows[b][t, sl] * EMBED_SCALE + pos_v[t, sl]
            return c2

        lax.fori_loop(0, SEQ_LEN, row_body, 0, unroll=4)

    gather(0, 0, True)

    def outer_body(gg, carry):
        @pl.when(gg >= 1)
        def _():
            out_copy(2 * gg - 1, 1, False)
        gather(2 * gg + 1, 1, True)
        gather(2 * gg, 0, False)
        compute(0)
        out_copy(2 * gg, 0, True)

        out_copy(2 * gg, 0, False)

        @pl.when(gg < SEQ_PER_WORKER // 2 - 1)
        def _():
            gather(2 * gg + 2, 0, True)
        gather(2 * gg + 1, 1, False)
        compute(1)
        out_copy(2 * gg + 1, 1, True)
        return carry

    lax.fori_loop(0, SEQ_PER_WORKER // 2, outer_body, 0)

    out_copy(SEQ_PER_WORKER - 1, 1, False)


@jax.jit
def _embed(inputs_flat, tok_table, pos_table):
    mesh = plsc.VectorSubcoreMesh(core_axis_name="c", subcore_axis_name="s")
    fn = pl.kernel(
        _sc_kernel_body,
        out_type=jax.ShapeDtypeStruct((BATCH * SEQ_LEN, EMBED_DIM),
                                      jnp.float32),
        mesh=mesh,
        scratch_types=[
            pltpu.VMEM((ROWS_PER_WORKER,), jnp.int32),
            pltpu.VMEM((SEQ_LEN, EMBED_DIM), jnp.float32),
            pltpu.VMEM((SEQ_LEN, EMBED_DIM), jnp.float32),
            pltpu.VMEM((SEQ_LEN, EMBED_DIM), jnp.float32),
            pltpu.SemaphoreType.DMA,
            pltpu.SemaphoreType.DMA,
            pltpu.SemaphoreType.DMA,
            pltpu.SemaphoreType.DMA,
        ],
        compiler_params=pltpu.CompilerParams(use_tc_tiling_on_sc=False),
    )
    return fn(inputs_flat, tok_table, pos_table)


def kernel(inputs, tok_table, pos_table):
    flat = inputs.reshape(BATCH * SEQ_LEN)
    out = _embed(flat, tok_table, pos_table)
    return out.reshape(BATCH, SEQ_LEN, EMBED_DIM)


# speedup vs baseline: 1.3440x; 1.1439x over previous
import jax
import jax.numpy as jnp
from jax import lax
from jax.experimental import pallas as pl
from jax.experimental.pallas import tpu as pltpu
from jax.experimental.pallas import tpu_sc as plsc

VOCAB = 1000000
SEQ_LEN = 200
EMBED_DIM = 64
BATCH = 4096

NUM_CORES = 2
NUM_SUBCORES = 16
NUM_WORKERS = NUM_CORES * NUM_SUBCORES
SEQ_PER_WORKER = BATCH // NUM_WORKERS
LANES = 16
VREGS_PER_ROW = EMBED_DIM // LANES

SEQ_PER_CHUNK = 2
ROWS_PER_CHUNK = SEQ_PER_CHUNK * SEQ_LEN
NUM_CHUNKS = SEQ_PER_WORKER // SEQ_PER_CHUNK
NBUF = 4

GATHER_SPLITS = ((0, 128), (128, 72))

EMBED_SCALE = 8.0


def _sc_kernel_body(inputs_hbm, tok_hbm, pos_hbm, out_hbm,
                    idx0, idx1, idx2, idx3,
                    rows0, rows1, rows2, rows3, pos_v,
                    sg0, sg1, sg2, sg3, so0, so1, so2, so3,
                    si0, si1, si2, si3):
    wid = lax.axis_index("s") * NUM_CORES + lax.axis_index("c")
    base_seq = wid * SEQ_PER_WORKER

    idx = (idx0, idx1, idx2, idx3)
    rows = (rows0, rows1, rows2, rows3)
    sem_g = (sg0, sg1, sg2, sg3)
    sem_o = (so0, so1, so2, so3)
    sem_i = (si0, si1, si2, si3)

    pltpu.sync_copy(pos_hbm, pos_v)

    def idx_io(g, b, start):
        desc = pltpu.make_async_copy(
            inputs_hbm.at[pl.ds(base_seq + g * SEQ_PER_CHUNK, SEQ_PER_CHUNK)],
            idx[b], sem_i[b])
        desc.start() if start else desc.wait()

    def gather_io(b, start):
        for s in range(SEQ_PER_CHUNK):
            for (off, n) in GATHER_SPLITS:
                desc = pltpu.make_async_copy(
                    tok_hbm.at[idx[b].at[s, pl.ds(off, n)]],
                    rows[b].at[pl.ds(s * SEQ_LEN + off, n)],
                    sem_g[b])
                desc.start() if start else desc.wait()

    def out_io(g, b, start):
        for s in range(SEQ_PER_CHUNK):
            desc = pltpu.make_async_copy(
                rows[b].at[pl.ds(s * SEQ_LEN, SEQ_LEN)],
                out_hbm.at[base_seq + g * SEQ_PER_CHUNK + s],
                sem_o[b])
            desc.start() if start else desc.wait()

    def compute(b):
        def row_body(t, c2):
            for c in range(VREGS_PER_ROW):
                sl = pl.ds(c * LANES, LANES)
                p = pos_v[t, sl]
                for s in range(SEQ_PER_CHUNK):
                    r = s * SEQ_LEN + t
                    rows[b][r, sl] = rows[b][r, sl] * EMBED_SCALE + p
            return c2

        lax.fori_loop(0, SEQ_LEN, row_body, 0, unroll=2)

    for j in range(NBUF):
        idx_io(j, j, True)
    for j in range(NBUF - 1):
        idx_io(j, j, False)
        gather_io(j, True)

    def outer_body(gg, carry):
        for b in range(NBUF):
            g = gg * NBUF + b
            nb = (b + 3) % NBUF

            @pl.when(g >= 1)
            def _():
                out_io(g - 1, nb, False)

            @pl.when(g + 3 < NUM_CHUNKS)
            def _():
                idx_io(g + 3, nb, False)
                gather_io(nb, True)

            gather_io(b, False)

            @pl.when(g + 4 < NUM_CHUNKS)
            def _():
                idx_io(g + 4, b, True)

            compute(b)
            out_io(g, b, True)
        return carry

    lax.fori_loop(0, NUM_CHUNKS // NBUF, outer_body, 0)

    out_io(NUM_CHUNKS - 1, 3, False)


@jax.jit
def _embed(inputs, tok_table, pos_table):
    mesh = plsc.VectorSubcoreMesh(core_axis_name="c", subcore_axis_name="s")
    fn = pl.kernel(
        _sc_kernel_body,
        out_type=jax.ShapeDtypeStruct((BATCH, SEQ_LEN, EMBED_DIM),
                                      jnp.float32),
        mesh=mesh,
        scratch_types=(
            [pltpu.VMEM((SEQ_PER_CHUNK, SEQ_LEN), jnp.int32)] * NBUF
            + [pltpu.VMEM((ROWS_PER_CHUNK, EMBED_DIM), jnp.float32)] * NBUF
            + [pltpu.VMEM((SEQ_LEN, EMBED_DIM), jnp.float32)]
            + [pltpu.SemaphoreType.DMA] * (3 * NBUF)
        ),
        compiler_params=pltpu.CompilerParams(use_tc_tiling_on_sc=False),
    )
    return fn(inputs, tok_table, pos_table)


def kernel(inputs, tok_table, pos_table):
    return _embed(inputs, tok_table, pos_table)


# speedup vs baseline: 1.7177x; 1.2781x over previous
import jax
import jax.numpy as jnp
from jax import lax
from jax.experimental import pallas as pl
from jax.experimental.pallas import tpu as pltpu
from jax.experimental.pallas import tpu_sc as plsc

VOCAB = 1000000
SEQ_LEN = 200
EMBED_DIM = 64
BATCH = 4096

NUM_CORES = 2
NUM_SUBCORES = 16
NUM_WORKERS = NUM_CORES * NUM_SUBCORES
SEQ_PER_WORKER = BATCH // NUM_WORKERS
LANES = 16
VREGS_PER_ROW = EMBED_DIM // LANES

SEQ_PER_CHUNK = 2
ROWS_PER_CHUNK = SEQ_PER_CHUNK * SEQ_LEN
NUM_CHUNKS = SEQ_PER_WORKER // SEQ_PER_CHUNK
NBUF = 4

GATHER_SPLITS = ((0, 128), (128, 72))

EMBED_SCALE = 8.0


def _sc_kernel_body(inputs_hbm, tok_hbm, pos_hbm, out_hbm,
                    idx0, idx1, idx2, idx3,
                    rows0, rows1, rows2, rows3, pos_v,
                    sg0, sg1, sg2, sg3, so0, so1, so2, so3,
                    si0, si1, si2, si3):
    wid = lax.axis_index("s") * NUM_CORES + lax.axis_index("c")
    base_seq = wid * SEQ_PER_WORKER

    idx = (idx0, idx1, idx2, idx3)
    rows = (rows0, rows1, rows2, rows3)
    sem_g = (sg0, sg1, sg2, sg3)
    sem_o = (so0, so1, so2, so3)
    sem_i = (si0, si1, si2, si3)

    pltpu.sync_copy(pos_hbm, pos_v)

    def idx_io(g, b, start):
        desc = pltpu.make_async_copy(
            inputs_hbm.at[pl.ds(base_seq + g * SEQ_PER_CHUNK, SEQ_PER_CHUNK)],
            idx[b], sem_i[b])
        desc.start() if start else desc.wait()

    def gather_io(b, start):
        for s in range(SEQ_PER_CHUNK):
            for (off, n) in GATHER_SPLITS:
                desc = pltpu.make_async_copy(
                    tok_hbm.at[idx[b].at[s, pl.ds(off, n)]],
                    rows[b].at[pl.ds(s * SEQ_LEN + off, n)],
                    sem_g[b])
                desc.start() if start else desc.wait()

    def out_io(g, b, start):
        for s in range(SEQ_PER_CHUNK):
            desc = pltpu.make_async_copy(
                rows[b].at[pl.ds(s * SEQ_LEN, SEQ_LEN)],
                out_hbm.at[base_seq + g * SEQ_PER_CHUNK + s, :,
                           pl.ds(0, EMBED_DIM)],
                sem_o[b])
            desc.start() if start else desc.wait()

    def compute(b):
        def row_body(t, c2):
            for c in range(VREGS_PER_ROW):
                sl = pl.ds(c * LANES, LANES)
                p = pos_v[t, sl]
                for s in range(SEQ_PER_CHUNK):
                    r = s * SEQ_LEN + t
                    rows[b][r, sl] = rows[b][r, sl] * EMBED_SCALE + p
            return c2

        lax.fori_loop(0, SEQ_LEN, row_body, 0, unroll=2)

    for j in range(NBUF):
        idx_io(j, j, True)
    for j in range(NBUF - 1):
        idx_io(j, j, False)
        gather_io(j, True)

    def outer_body(gg, carry):
        for b in range(NBUF):
            g = gg * NBUF + b
            nb = (b + 3) % NBUF

            @pl.when(g >= 1)
            def _():
                out_io(g - 1, nb, False)

            @pl.when(g + 3 < NUM_CHUNKS)
            def _():
                idx_io(g + 3, nb, False)
                gather_io(nb, True)

            gather_io(b, False)

            @pl.when(g + 4 < NUM_CHUNKS)
            def _():
                idx_io(g + 4, b, True)

            compute(b)
            out_io(g, b, True)
        return carry

    lax.fori_loop(0, NUM_CHUNKS // NBUF, outer_body, 0)

    out_io(NUM_CHUNKS - 1, 3, False)


@jax.jit
def _embed(inputs, tok_table, pos_table):
    mesh = plsc.VectorSubcoreMesh(core_axis_name="c", subcore_axis_name="s")
    fn = pl.kernel(
        _sc_kernel_body,
        out_type=jax.ShapeDtypeStruct((BATCH, SEQ_LEN, 2 * EMBED_DIM),
                                      jnp.float32),
        mesh=mesh,
        scratch_types=(
            [pltpu.VMEM((SEQ_PER_CHUNK, SEQ_LEN), jnp.int32)] * NBUF
            + [pltpu.VMEM((ROWS_PER_CHUNK, EMBED_DIM), jnp.float32)] * NBUF
            + [pltpu.VMEM((SEQ_LEN, EMBED_DIM), jnp.float32)]
            + [pltpu.SemaphoreType.DMA] * (3 * NBUF)
        ),
        compiler_params=pltpu.CompilerParams(use_tc_tiling_on_sc=False),
    )
    return fn(inputs, tok_table, pos_table)


def kernel(inputs, tok_table, pos_table):
    return _embed(inputs, tok_table, pos_table)[:, :, :EMBED_DIM]
